# baseline (device time: 121921 ns/iter reference)
import jax
import jax.numpy as jnp
from jax import lax
from jax.experimental import pallas as pl
from jax.experimental.pallas import tpu as pltpu


def kernel(ids, E):
    V, D = E.shape
    T = ids.shape[0]

    z = lax.axis_index("z")
    local = ids - z * V
    valid = (local >= 0) & (local < V)
    rows = jnp.take(E, jnp.clip(local, 0, V - 1), axis=0)
    partial = jnp.where(valid[:, None], rows, 0.0).astype(jnp.bfloat16)

    def body(p_ref, out_ref, recv_ref, send_sem, recv_sem):
        x = lax.axis_index("x")
        y = lax.axis_index("y")
        zz = lax.axis_index("z")
        peer = (x, y, 1 - zz)

        barrier = pltpu.get_barrier_semaphore()
        pl.semaphore_signal(
            barrier, inc=1, device_id=peer, device_id_type=pl.DeviceIdType.MESH
        )
        pl.semaphore_wait(barrier, 1)

        rdma = pltpu.make_async_remote_copy(
            src_ref=p_ref,
            dst_ref=recv_ref,
            send_sem=send_sem,
            recv_sem=recv_sem,
            device_id=peer,
            device_id_type=pl.DeviceIdType.MESH,
        )
        rdma.start()
        rdma.wait()

        out_ref[...] = p_ref[...].astype(jnp.float32) + recv_ref[...].astype(
            jnp.float32
        )

    return pl.pallas_call(
        body,
        out_shape=jax.ShapeDtypeStruct((T, D), jnp.float32),
        in_specs=[pl.BlockSpec(memory_space=pltpu.VMEM)],
        out_specs=pl.BlockSpec(memory_space=pltpu.VMEM),
        scratch_shapes=[
            pltpu.VMEM((T, D), jnp.bfloat16),
            pltpu.SemaphoreType.DMA,
            pltpu.SemaphoreType.DMA,
        ],
        compiler_params=pltpu.CompilerParams(collective_id=0),
    )(partial)


# device time: 98276 ns/iter; 1.2406x vs baseline; 1.2406x over previous
import jax
import jax.numpy as jnp
from jax import lax
from jax.experimental import pallas as pl
from jax.experimental.pallas import tpu as pltpu

UNROLL = 8


def kernel(ids, E):
    V, D = E.shape
    T = ids.shape[0]
    ids_col = ids.reshape(T, 1)

    def body(
        ids_smem,
        ids_col_ref,
        e_hbm,
        out_ref,
        gather,
        partial,
        recv,
        gsem,
        send_sem,
        recv_sem,
    ):
        x = lax.axis_index("x")
        y = lax.axis_index("y")
        z = lax.axis_index("z")
        peer = (x, y, 1 - z)

        barrier = pltpu.get_barrier_semaphore()
        pl.semaphore_signal(
            barrier, inc=1, device_id=peer, device_id_type=pl.DeviceIdType.MESH
        )
        pl.semaphore_wait(barrier, 1)

        base = z * V

        def issue(i, _):
            for u in range(UNROLL):
                t = i * UNROLL + u
                idx = jnp.clip(ids_smem[t] - base, 0, V - 1)
                pltpu.make_async_copy(
                    e_hbm.at[pl.ds(idx, 1), :], gather.at[pl.ds(t, 1), :], gsem
                ).start()
            return 0

        lax.fori_loop(0, T // UNROLL, issue, 0)

        def drain(i, _):
            for u in range(UNROLL):
                t = i * UNROLL + u
                pltpu.make_async_copy(
                    e_hbm.at[pl.ds(0, 1), :], gather.at[pl.ds(t, 1), :], gsem
                ).wait()
            return 0

        lax.fori_loop(0, T // UNROLL, drain, 0)

        valid = (ids_col_ref[...] >= base) & (ids_col_ref[...] < base + V)
        partial[...] = jnp.where(valid, gather[...], 0.0).astype(jnp.bfloat16)

        rdma = pltpu.make_async_remote_copy(
            src_ref=partial,
            dst_ref=recv,
            send_sem=send_sem,
            recv_sem=recv_sem,
            device_id=peer,
            device_id_type=pl.DeviceIdType.MESH,
        )
        rdma.start()
        rdma.wait()

        out_ref[...] = partial[...].astype(jnp.float32) + recv[...].astype(
            jnp.float32
        )

    return pl.pallas_call(
        body,
        out_shape=jax.ShapeDtypeStruct((T, D), jnp.float32),
        in_specs=[
            pl.BlockSpec(memory_space=pltpu.SMEM),
            pl.BlockSpec(memory_space=pltpu.VMEM),
            pl.BlockSpec(memory_space=pl.ANY),
        ],
        out_specs=pl.BlockSpec(memory_space=pltpu.VMEM),
        scratch_shapes=[
            pltpu.VMEM((T, D), jnp.float32),
            pltpu.VMEM((T, D), jnp.bfloat16),
            pltpu.VMEM((T, D), jnp.bfloat16),
            pltpu.SemaphoreType.DMA,
            pltpu.SemaphoreType.DMA,
            pltpu.SemaphoreType.DMA,
        ],
        compiler_params=pltpu.CompilerParams(collective_id=0),
    )(ids, ids_col, E)


# device time: 55440 ns/iter; 2.1992x vs baseline; 1.7727x over previous
import jax
import jax.numpy as jnp
from jax import lax
from jax.experimental import pallas as pl
from jax.experimental.pallas import tpu as pltpu

UNROLL = 8
C = 8


def kernel(ids, E):
    V, D = E.shape
    T = ids.shape[0]
    H = T // 2
    S = H // C
    ids_col = ids.reshape(T, 1)

    def body(
        ids_smem,
        ids_col_ref,
        e_hbm,
        out_ref,
        gather,
        zsend,
        zrecv,
        xsend,
        xrecv,
        gsems,
        zsend_sems,
        zrecv_sems,
        xsend_sems,
        xrecv_sems,
    ):
        x = lax.axis_index("x")
        y = lax.axis_index("y")
        z = lax.axis_index("z")
        peer_z = (x, y, 1 - z)
        peer_x = (1 - x, y, z)

        barrier = pltpu.get_barrier_semaphore()
        for peer in (peer_z, peer_x):
            pl.semaphore_signal(
                barrier, inc=1, device_id=peer, device_id_type=pl.DeviceIdType.MESH
            )
        pl.semaphore_wait(barrier, 2)

        base = z * V
        h0 = x * H
        g0 = (1 - x) * H

        def issue_gather(c):
            def step(i, _):
                for u in range(UNROLL):
                    r = c * S + i * UNROLL + u
                    idx = jnp.clip(ids_smem[h0 + r] - base, 0, V - 1)
                    pltpu.make_async_copy(
                        e_hbm.at[pl.ds(idx, 1), :],
                        gather.at[pl.ds(r, 1), :],
                        gsems.at[c],
                    ).start()
                return 0

            lax.fori_loop(0, S // UNROLL, step, 0)

        def wait_gather(c):
            def step(i, _):
                for u in range(UNROLL):
                    r = c * S + i * UNROLL + u
                    pltpu.make_async_copy(
                        e_hbm.at[pl.ds(0, 1), :],
                        gather.at[pl.ds(r, 1), :],
                        gsems.at[c],
                    ).wait()
                return 0

            lax.fori_loop(0, S // UNROLL, step, 0)

        def z_rdma(c):
            sl = pl.ds(c * S, S)
            return pltpu.make_async_remote_copy(
                src_ref=zsend.at[sl],
                dst_ref=zrecv.at[sl],
                send_sem=zsend_sems.at[c],
                recv_sem=zrecv_sems.at[c],
                device_id=peer_z,
                device_id_type=pl.DeviceIdType.MESH,
            )

        def x_rdma(c):
            sl = pl.ds(c * S, S)
            return pltpu.make_async_remote_copy(
                src_ref=xsend.at[sl],
                dst_ref=xrecv.at[sl],
                send_sem=xsend_sems.at[c],
                recv_sem=xrecv_sems.at[c],
                device_id=peer_x,
                device_id_type=pl.DeviceIdType.MESH,
            )

        issue_gather(0)
        for c in range(C):
            if c + 1 < C:
                issue_gather(c + 1)
            wait_gather(c)
            sl = pl.ds(c * S, S)
            idc = ids_col_ref[pl.ds(h0 + c * S, S), :]
            valid = (idc >= base) & (idc < base + V)
            zsend[sl] = jnp.where(valid, gather[sl], 0.0).astype(jnp.bfloat16)
            z_rdma(c).start()

        for c in range(C):
            z_rdma(c).wait_recv()
            sl = pl.ds(c * S, S)
            red = zsend[sl].astype(jnp.float32) + zrecv[sl].astype(jnp.float32)
            out_ref[pl.ds(h0 + c * S, S), :] = red
            xsend[sl] = red.astype(jnp.bfloat16)
            x_rdma(c).start()

        for c in range(C):
            x_rdma(c).wait_recv()
            sl = pl.ds(c * S, S)
            out_ref[pl.ds(g0 + c * S, S), :] = xrecv[sl].astype(jnp.float32)

        for c in range(C):
            z_rdma(c).wait_send()
            x_rdma(c).wait_send()

    return pl.pallas_call(
        body,
        out_shape=jax.ShapeDtypeStruct((T, D), jnp.float32),
        in_specs=[
            pl.BlockSpec(memory_space=pltpu.SMEM),
            pl.BlockSpec(memory_space=pltpu.VMEM),
            pl.BlockSpec(memory_space=pl.ANY),
        ],
        out_specs=pl.BlockSpec(memory_space=pltpu.VMEM),
        scratch_shapes=[
            pltpu.VMEM((H, D), jnp.float32),
            pltpu.VMEM((H, D), jnp.bfloat16),
            pltpu.VMEM((H, D), jnp.bfloat16),
            pltpu.VMEM((H, D), jnp.bfloat16),
            pltpu.VMEM((H, D), jnp.bfloat16),
            pltpu.SemaphoreType.DMA((C,)),
            pltpu.SemaphoreType.DMA((C,)),
            pltpu.SemaphoreType.DMA((C,)),
            pltpu.SemaphoreType.DMA((C,)),
            pltpu.SemaphoreType.DMA((C,)),
        ],
        compiler_params=pltpu.CompilerParams(collective_id=0),
    )(ids, ids_col, E)
